# Initial kernel scaffold; baseline (speedup 1.0000x reference)
import functools

import jax
import jax.numpy as jnp
from jax import lax
from jax.experimental import pallas as pl
from jax.experimental.pallas import tpu as pltpu
from jax.experimental.pallas import tpu_sc as plsc

NUM_DFAS = 128
NUM_STATES = 64
ALPHABET = 128
BATCH = 2048
SEQ_LEN = 256

NC = 2
NS = 16
L = 16
NW = NC * NS
DPW = NUM_DFAS // NW
BC = 256
NCHUNK = BATCH // BC
NGROUP = BC // L


def _dfa_body(xT_h, init_h, acc_h, tab_h, out_h, table_v, acc_v, init_v,
              xbuf, out_v):
    wid = lax.axis_index("s") * NC + lax.axis_index("c")
    d0 = wid * DPW

    pltpu.sync_copy(tab_h.at[pl.ds(d0 * NUM_STATES * ALPHABET,
                                   DPW * NUM_STATES * ALPHABET)], table_v)
    pltpu.sync_copy(acc_h.at[pl.ds(d0 * NUM_STATES, DPW * NUM_STATES)], acc_v)
    pltpu.sync_copy(init_h, init_v)

    si = [plsc.load_gather(init_v, [jnp.full((L,), d0 + j, jnp.int32)])
          for j in range(DPW)]
    base = [jnp.full((L,), j * NUM_STATES * ALPHABET, jnp.int32)
            for j in range(DPW)]
    abase = [jnp.full((L,), j * NUM_STATES, jnp.int32) for j in range(DPW)]

    for chunk in range(NCHUNK):
        b0 = chunk * BC
        pltpu.sync_copy(xT_h.at[:, pl.ds(b0, BC)], xbuf)

        def bg_body(bg, carry):
            def t_body(t, states):
                c = xbuf[t, pl.ds(bg * L, L)]
                return tuple(
                    plsc.load_gather(
                        table_v, [base[j] + states[j] * ALPHABET + c])
                    for j in range(DPW))

            states = lax.fori_loop(0, SEQ_LEN, t_body, tuple(si))
            for j in range(DPW):
                a = plsc.load_gather(acc_v, [abase[j] + states[j]])
                out_v[j, pl.ds(bg * L, L)] = a
            return carry

        lax.fori_loop(0, NGROUP, bg_body, 0)
        pltpu.sync_copy(out_v, out_h.at[pl.ds(d0, DPW), pl.ds(b0, BC)])


@jax.jit
def _run(xT, init, acc_i32, tab_flat):
    mesh = plsc.VectorSubcoreMesh(core_axis_name="c", subcore_axis_name="s",
                                  num_cores=NC, num_subcores=NS)
    f = pl.kernel(
        _dfa_body,
        out_type=jax.ShapeDtypeStruct((NUM_DFAS, BATCH), jnp.int32),
        mesh=mesh,
        scratch_types=[
            pltpu.VMEM((DPW * NUM_STATES * ALPHABET,), jnp.int32),
            pltpu.VMEM((DPW * NUM_STATES,), jnp.int32),
            pltpu.VMEM((NUM_DFAS,), jnp.int32),
            pltpu.VMEM((SEQ_LEN, BC), jnp.int32),
            pltpu.VMEM((DPW, BC), jnp.int32),
        ],
    )
    return f(xT, init, acc_i32, tab_flat)


def kernel(x, initial_state, accepting_states, transition_function):
    xT = x.T.reshape(SEQ_LEN, BATCH)
    acc_i32 = accepting_states.astype(jnp.int32).reshape(-1)
    tab_flat = transition_function.reshape(-1)
    out = _run(xT, initial_state, acc_i32, tab_flat)
    return out.astype(jnp.bool_)


# speedup vs baseline: 18.5528x; 18.5528x over previous
import functools

import jax
import jax.numpy as jnp
from jax import lax
from jax.experimental import pallas as pl
from jax.experimental.pallas import tpu as pltpu
from jax.experimental.pallas import tpu_sc as plsc

NUM_DFAS = 128
NUM_STATES = 64
ALPHABET = 128
BATCH = 2048
SEQ_LEN = 256

NC = 2
NS = 16
L = 16
NW = NC * NS
DPW = NUM_DFAS // NW
BC = 256
NCHUNK = BATCH // BC
NGROUP = BC // L


def _dfa_body(xT_h, init_h, acc_h, tab_h, out_h, table_v, acc_v, init_v,
              xbuf, out_v):
    wid = lax.axis_index("s") * NC + lax.axis_index("c")
    d0 = wid * DPW

    pltpu.sync_copy(tab_h.at[pl.ds(d0 * NUM_STATES * ALPHABET,
                                   DPW * NUM_STATES * ALPHABET)], table_v)
    pltpu.sync_copy(acc_h.at[pl.ds(d0 * NUM_STATES, DPW * NUM_STATES)], acc_v)
    pltpu.sync_copy(init_h, init_v)

    si = [plsc.load_gather(init_v, [jnp.full((L,), d0 + j, jnp.int32)])
          for j in range(DPW)]
    base = [jnp.full((L,), j * NUM_STATES * ALPHABET, jnp.int32)
            for j in range(DPW)]
    abase = [jnp.full((L,), j * NUM_STATES, jnp.int32) for j in range(DPW)]

    for chunk in range(NCHUNK):
        b0 = chunk * BC
        pltpu.sync_copy(xT_h.at[:, pl.ds(b0, BC)], xbuf)

        def bg_body(bg, carry):
            def t_body(t, states):
                c = xbuf[t, pl.ds(bg * L, L)]
                return tuple(
                    plsc.load_gather(
                        table_v, [base[j] + states[j] * ALPHABET + c])
                    for j in range(DPW))

            states = lax.fori_loop(0, SEQ_LEN, t_body, tuple(si))
            for j in range(DPW):
                a = plsc.load_gather(acc_v, [abase[j] + states[j]])
                out_v[j, pl.ds(bg * L, L)] = a
            return carry

        lax.fori_loop(0, NGROUP, bg_body, 0)
        pltpu.sync_copy(out_v, out_h.at[pl.ds(d0, DPW), pl.ds(b0, BC)])


@jax.jit
def _run(xT, init, acc_i32, tab_flat):
    mesh = plsc.VectorSubcoreMesh(core_axis_name="c", subcore_axis_name="s",
                                  num_cores=NC, num_subcores=NS)
    f = pl.kernel(
        _dfa_body,
        out_type=jax.ShapeDtypeStruct((NUM_DFAS, BATCH), jnp.int32),
        mesh=mesh,
        scratch_types=[
            pltpu.VMEM((DPW * NUM_STATES * ALPHABET,), jnp.int32),
            pltpu.VMEM((DPW * NUM_STATES,), jnp.int32),
            pltpu.VMEM((NUM_DFAS,), jnp.int32),
            pltpu.VMEM((SEQ_LEN, BC), jnp.int32),
            pltpu.VMEM((DPW, BC), jnp.int32),
        ],
        compiler_params=pltpu.CompilerParams(needs_layout_passes=False),
    )
    return f(xT, init, acc_i32, tab_flat)


def kernel(x, initial_state, accepting_states, transition_function):
    xT = x.T.reshape(SEQ_LEN, BATCH)
    acc_i32 = accepting_states.astype(jnp.int32).reshape(-1)
    tab_flat = transition_function.reshape(-1)
    out = _run(xT, initial_state, acc_i32, tab_flat)
    return out.astype(jnp.bool_)


# speedup vs baseline: 31.8954x; 1.7192x over previous
import jax
import jax.numpy as jnp
from jax import lax
from jax.experimental import pallas as pl
from jax.experimental.pallas import tpu as pltpu
from jax.experimental.pallas import tpu_sc as plsc

NUM_DFAS = 128
NUM_STATES = 64
ALPHABET = 128
BATCH = 2048
SEQ_LEN = 256

NC = 2
NS = 16
L = 16
NW = NC * NS
DPW = NUM_DFAS // NW
TSIZE = NUM_STATES * ALPHABET
BC = 256
NCHUNK = BATCH // BC
NPAIR = BC // (2 * L)


def _dfa_body(xT_h, init_h, acc_h, tab_h, out_h, table_v, acc_v, init_v,
              xbuf, out_v):
    wid = lax.axis_index("s") * NC + lax.axis_index("c")
    d0 = wid * DPW

    pltpu.sync_copy(tab_h.at[pl.ds(d0 * TSIZE, DPW * TSIZE)], table_v)
    pltpu.sync_copy(acc_h.at[pl.ds(d0 * NUM_STATES, DPW * NUM_STATES)], acc_v)
    pltpu.sync_copy(init_h, init_v)

    si = [plsc.load_gather(init_v, [jnp.full((L,), d0 + j, jnp.int32)])
          * ALPHABET + j * TSIZE
          for j in range(DPW)]
    si = tuple(si + si)

    for chunk in range(NCHUNK):
        b0 = chunk * BC
        pltpu.sync_copy(xT_h.at[:, pl.ds(b0, BC)], xbuf)

        def bg_body(bg, carry):
            def t_body(t, states):
                c0 = xbuf[t, pl.ds(bg * 2 * L, L)]
                c1 = xbuf[t, pl.ds(bg * 2 * L + L, L)]
                return tuple(
                    plsc.load_gather(table_v, [states[k] + (c0 if k < DPW
                                                            else c1)])
                    for k in range(2 * DPW))

            states = lax.fori_loop(0, SEQ_LEN, t_body, si, unroll=2)
            for k in range(2 * DPW):
                a = plsc.load_gather(
                    acc_v, [lax.shift_right_logical(states[k], 7)])
                out_v[k % DPW, pl.ds(bg * 2 * L + (k // DPW) * L, L)] = a
            return carry

        lax.fori_loop(0, NPAIR, bg_body, 0)
        pltpu.sync_copy(out_v, out_h.at[pl.ds(d0, DPW), pl.ds(b0, BC)])


@jax.jit
def _run(xT, init, acc_i32, tab_scaled):
    mesh = plsc.VectorSubcoreMesh(core_axis_name="c", subcore_axis_name="s",
                                  num_cores=NC, num_subcores=NS)
    f = pl.kernel(
        _dfa_body,
        out_type=jax.ShapeDtypeStruct((NUM_DFAS, BATCH), jnp.int32),
        mesh=mesh,
        scratch_types=[
            pltpu.VMEM((DPW * TSIZE,), jnp.int32),
            pltpu.VMEM((DPW * NUM_STATES,), jnp.int32),
            pltpu.VMEM((NUM_DFAS,), jnp.int32),
            pltpu.VMEM((SEQ_LEN, BC), jnp.int32),
            pltpu.VMEM((DPW, BC), jnp.int32),
        ],
        compiler_params=pltpu.CompilerParams(needs_layout_passes=False),
    )
    return f(xT, init, acc_i32, tab_scaled)


def kernel(x, initial_state, accepting_states, transition_function):
    xT = x.T.reshape(SEQ_LEN, BATCH)
    acc_i32 = accepting_states.astype(jnp.int32).reshape(-1)
    dlocal = (jnp.arange(NUM_DFAS, dtype=jnp.int32) % DPW)[:, None, None]
    tab_scaled = (transition_function * ALPHABET + dlocal * TSIZE).reshape(-1)
    out = _run(xT, initial_state, acc_i32, tab_scaled)
    return out.astype(jnp.bool_)


# speedup vs baseline: 36.1308x; 1.1328x over previous
import jax
import jax.numpy as jnp
from jax import lax
from jax.experimental import pallas as pl
from jax.experimental.pallas import tpu as pltpu
from jax.experimental.pallas import tpu_sc as plsc

NUM_DFAS = 128
NUM_STATES = 64
ALPHABET = 128
BATCH = 2048
SEQ_LEN = 256

NC = 2
NS = 16
L = 16
NW = NC * NS
DPW = NUM_DFAS // NW
TSIZE = NUM_STATES * ALPHABET
BC = 256
NCHUNK = BATCH // BC
GROUPS = 4
NCHAIN = GROUPS * DPW
NQUAD = BC // (GROUPS * L)


def _dfa_body(xT_h, init_h, acc_h, tab_h, out_h, table_v, acc_v, init_v,
              xbuf, out_v):
    wid = lax.axis_index("s") * NC + lax.axis_index("c")
    d0 = wid * DPW

    pltpu.sync_copy(tab_h.at[pl.ds(d0 * TSIZE, DPW * TSIZE)], table_v)
    pltpu.sync_copy(acc_h.at[pl.ds(d0 * NUM_STATES, DPW * NUM_STATES)], acc_v)
    pltpu.sync_copy(init_h, init_v)

    si = [plsc.load_gather(init_v, [jnp.full((L,), d0 + j, jnp.int32)])
          * ALPHABET + j * TSIZE
          for j in range(DPW)]
    si = tuple(si * GROUPS)

    for chunk in range(NCHUNK):
        b0 = chunk * BC
        pltpu.sync_copy(xT_h.at[:, pl.ds(b0, BC)], xbuf)

        def bg_body(bg, carry):
            def t_body(t, states):
                c = [xbuf[t, pl.ds(bg * GROUPS * L + g * L, L)]
                     for g in range(GROUPS)]
                return tuple(
                    plsc.load_gather(table_v, [states[k] + c[k // DPW]])
                    for k in range(NCHAIN))

            states = lax.fori_loop(0, SEQ_LEN, t_body, si, unroll=2)
            for k in range(NCHAIN):
                a = plsc.load_gather(
                    acc_v, [lax.shift_right_logical(states[k], 7)])
                out_v[k % DPW,
                      pl.ds(bg * GROUPS * L + (k // DPW) * L, L)] = a
            return carry

        lax.fori_loop(0, NQUAD, bg_body, 0)
        pltpu.sync_copy(out_v, out_h.at[pl.ds(d0, DPW), pl.ds(b0, BC)])


@jax.jit
def _run(xT, init, acc_i32, tab_scaled):
    mesh = plsc.VectorSubcoreMesh(core_axis_name="c", subcore_axis_name="s",
                                  num_cores=NC, num_subcores=NS)
    f = pl.kernel(
        _dfa_body,
        out_type=jax.ShapeDtypeStruct((NUM_DFAS, BATCH), jnp.int32),
        mesh=mesh,
        scratch_types=[
            pltpu.VMEM((DPW * TSIZE,), jnp.int32),
            pltpu.VMEM((DPW * NUM_STATES,), jnp.int32),
            pltpu.VMEM((NUM_DFAS,), jnp.int32),
            pltpu.VMEM((SEQ_LEN, BC), jnp.int32),
            pltpu.VMEM((DPW, BC), jnp.int32),
        ],
        compiler_params=pltpu.CompilerParams(needs_layout_passes=False),
    )
    return f(xT, init, acc_i32, tab_scaled)


def kernel(x, initial_state, accepting_states, transition_function):
    xT = x.T.reshape(SEQ_LEN, BATCH)
    acc_i32 = accepting_states.astype(jnp.int32).reshape(-1)
    dlocal = (jnp.arange(NUM_DFAS, dtype=jnp.int32) % DPW)[:, None, None]
    tab_scaled = (transition_function * ALPHABET + dlocal * TSIZE).reshape(-1)
    out = _run(xT, initial_state, acc_i32, tab_scaled)
    return out.astype(jnp.bool_)
